# baseline (speedup 1.0000x reference)
import functools

import jax
import jax.numpy as jnp
from jax import lax
from jax.experimental import pallas as pl
from jax.experimental.pallas import tpu as pltpu
from jax.experimental.pallas import tpu_sc as plsc

_DIM = 128
_NROWS = 11
_NC, _NS = 2, 16
_NW = _NC * _NS
_BLK = 128
_LANES = 16


def _make_xe_kernel(n_edges):
    nblk = n_edges // _BLK
    jmax = -(-nblk // _NW)
    ngrp = _BLK // _LANES

    mesh = plsc.VectorSubcoreMesh(
        core_axis_name="c", subcore_axis_name="s",
        num_cores=_NC, num_subcores=_NS,
    )

    @functools.partial(
        pl.kernel,
        out_type=jax.ShapeDtypeStruct((n_edges * _DIM,), jnp.float32),
        mesh=mesh,
        compiler_params=pltpu.CompilerParams(needs_layout_passes=False),
        scratch_types=[
            pltpu.VMEM((_NROWS * _DIM,), jnp.float32),
            pltpu.VMEM((_BLK,), jnp.int32),
            pltpu.VMEM((_BLK * _DIM,), jnp.float32),
        ],
    )
    def xe_kernel(x_e, orders, btab, out, btab_v, idx_v, buf):
        wid = lax.axis_index("s") * _NC + lax.axis_index("c")
        pltpu.sync_copy(btab, btab_v)
        lane = lax.iota(jnp.int32, _LANES)
        rowbase = [(g * _LANES + lane) * _DIM for g in range(ngrp)]

        @pl.loop(0, jmax)
        def _(j):
            bid = wid + _NW * j

            @pl.when(bid < nblk)
            def _():
                base = bid * _BLK
                pltpu.sync_copy(orders.at[pl.ds(base, _BLK)], idx_v)
                pltpu.sync_copy(x_e.at[pl.ds(base * _DIM, _BLK * _DIM)], buf)

                obase = [idx_v[pl.ds(g * _LANES, _LANES)] * _DIM
                         for g in range(ngrp)]

                @pl.loop(0, _DIM, unroll=2)
                def _(c):
                    for g in range(ngrp):
                        bias = plsc.load_gather(btab_v, [obase[g] + c])
                        plsc.addupdate_scatter(buf, [rowbase[g] + c], bias)

                pltpu.sync_copy(buf, out.at[pl.ds(base * _DIM, _BLK * _DIM)])

    return xe_kernel


def _xv_body(xv_ref, b_ref, out_ref):
    out_ref[...] = xv_ref[...] + b_ref[1:2, :]


def _xv_add(x_v, b_table):
    n = x_v.shape[0]
    blk = 2000
    return pl.pallas_call(
        _xv_body,
        out_shape=jax.ShapeDtypeStruct((n, _DIM), jnp.float32),
        in_specs=[
            pl.BlockSpec((blk, _DIM), lambda i: (i, 0)),
            pl.BlockSpec((_NROWS, _DIM), lambda i: (0, 0)),
        ],
        out_specs=pl.BlockSpec((blk, _DIM), lambda i: (i, 0)),
        grid=(n // blk,),
    )(x_v, b_table)


def kernel(x_v, x_e, edge_orders, b_table):
    n_edges = x_e.shape[0]
    xe_flat = _make_xe_kernel(n_edges)(
        x_e.reshape(-1), edge_orders, b_table.reshape(-1))
    xv_out = _xv_add(x_v, b_table)
    return (xv_out, xe_flat.reshape(n_edges, _DIM))


# speedup vs baseline: 3.2584x; 3.2584x over previous
import functools

import jax
import jax.numpy as jnp
from jax import lax
from jax.experimental import pallas as pl
from jax.experimental.pallas import tpu as pltpu
from jax.experimental.pallas import tpu_sc as plsc

_DIM = 128
_NROWS = 11
_NC, _NS = 2, 16
_NW = _NC * _NS
_BLK = 128
_LANES = 16


def _make_xe_kernel(n_edges):
    nblk = n_edges // _BLK
    jmax = -(-nblk // _NW)
    ngrp = _BLK // _LANES

    mesh = plsc.VectorSubcoreMesh(
        core_axis_name="c", subcore_axis_name="s",
        num_cores=_NC, num_subcores=_NS,
    )

    @functools.partial(
        pl.kernel,
        out_type=jax.ShapeDtypeStruct((n_edges * _DIM,), jnp.float32),
        mesh=mesh,
        compiler_params=pltpu.CompilerParams(needs_layout_passes=False),
        scratch_types=[
            pltpu.VMEM((_NROWS * _DIM,), jnp.float32),
            pltpu.VMEM((_BLK,), jnp.int32),
            pltpu.VMEM((_BLK * _DIM,), jnp.float32),
        ],
    )
    def xe_kernel(x_e, orders, btab, out, btab_v, idx_v, buf):
        wid = lax.axis_index("s") * _NC + lax.axis_index("c")
        pltpu.sync_copy(btab, btab_v)
        lane = lax.iota(jnp.int32, _LANES)
        colv = [v * _LANES + lane for v in range(_DIM // _LANES)]

        @pl.loop(0, jmax)
        def _(j):
            bid = wid + _NW * j

            @pl.when(bid < nblk)
            def _():
                base = bid * _BLK
                pltpu.sync_copy(orders.at[pl.ds(base, _BLK)], idx_v)
                pltpu.sync_copy(x_e.at[pl.ds(base * _DIM, _BLK * _DIM)], buf)

                @pl.loop(0, _BLK, unroll=4)
                def _(r):
                    sel = jnp.full((_LANES,), r, jnp.int32)
                    obc = plsc.load_gather(idx_v, [sel]) * _DIM
                    row0 = r * _DIM
                    for v in range(_DIM // _LANES):
                        bias = plsc.load_gather(btab_v, [obc + colv[v]])
                        plsc.addupdate(
                            buf.at[pl.ds(row0 + v * _LANES, _LANES)], bias)

                pltpu.sync_copy(buf, out.at[pl.ds(base * _DIM, _BLK * _DIM)])

    return xe_kernel


def _xv_body(xv_ref, b_ref, out_ref):
    out_ref[...] = xv_ref[...] + b_ref[1:2, :]


def _xv_add(x_v, b_table):
    n = x_v.shape[0]
    blk = 2000
    return pl.pallas_call(
        _xv_body,
        out_shape=jax.ShapeDtypeStruct((n, _DIM), jnp.float32),
        in_specs=[
            pl.BlockSpec((blk, _DIM), lambda i: (i, 0)),
            pl.BlockSpec((_NROWS, _DIM), lambda i: (0, 0)),
        ],
        out_specs=pl.BlockSpec((blk, _DIM), lambda i: (i, 0)),
        grid=(n // blk,),
    )(x_v, b_table)


def kernel(x_v, x_e, edge_orders, b_table):
    n_edges = x_e.shape[0]
    xe_flat = _make_xe_kernel(n_edges)(
        x_e.reshape(-1), edge_orders, b_table.reshape(-1))
    xv_out = _xv_add(x_v, b_table)
    return (xv_out, xe_flat.reshape(n_edges, _DIM))
